# baseline (speedup 1.0000x reference)
import functools

import jax
import jax.numpy as jnp
from jax import lax
from jax.experimental import pallas as pl
from jax.experimental.pallas import tpu as pltpu
from jax.experimental.pallas import tpu_sc as plsc

E = 8
K = 2
D = 1024
H = 4096
N = 2048
NK = N * K
BM = 256
P = NK + E * BM
NB = P // BM
NBP = 32
HT1 = 512
NH1 = H // HT1

NC = 2
NS = 16
NW = NC * NS


def _gelu_exact(v):
    return v * 0.5 * (1.0 + jax.lax.erf(v * (2.0 ** -0.5)))


def _router_body(x_ref, wg_ref, g_ref, e_ref):
    xv = x_ref[...]
    wg = wg_ref[0:E, :]
    logits = jax.lax.dot_general(xv, wg, (((1,), (1,)), ((), ())),
                                 preferred_element_type=jnp.float32)
    m = jnp.max(logits, axis=1, keepdims=True)
    p = jnp.exp(logits - m)
    p = p / jnp.sum(p, axis=1, keepdims=True)
    cols = jax.lax.broadcasted_iota(jnp.int32, p.shape, 1)
    m1 = jnp.max(p, axis=1, keepdims=True)
    i1 = jnp.min(jnp.where(p == m1, cols, E), axis=1, keepdims=True)
    mask1 = cols == i1
    p2 = jnp.where(mask1, -1.0, p)
    m2 = jnp.max(p2, axis=1, keepdims=True)
    i2 = jnp.min(jnp.where(p2 == m2, cols, E), axis=1, keepdims=True)
    g_ref[...] = jnp.concatenate([m1, m2], axis=1)
    e_ref[...] = jnp.concatenate([i1, i2], axis=1)


def _router(xf, Wg, interpret=False):
    return pl.pallas_call(
        _router_body,
        out_shape=(jax.ShapeDtypeStruct((N, K), jnp.float32),
                   jax.ShapeDtypeStruct((N, K), jnp.int32)),
        interpret=interpret,
    )(xf, Wg)


def _dispatch_body(e_hbm, dest_hbm, src_hbm, bexp_hbm,
                   e_v, dest_v, src_v, bexp_v, run_v, sem):
    cid = lax.axis_index("c")
    sid = lax.axis_index("s")

    @pl.when(jnp.logical_and(cid == 0, sid == 0))
    def _():
        pltpu.sync_copy(e_hbm, e_v)
        lanes = jax.lax.broadcasted_iota(jnp.int32, (16,), 0)
        zeros16 = jnp.zeros((16,), jnp.int32)

        def count_body(i, cnt):
            v = e_v[pl.ds(i * 16, 16)]
            for e in range(E):
                ce = jnp.sum(jnp.where(v == e, 1, 0))
                cnt = cnt + jnp.where(lanes == e, ce, 0)
            return cnt

        cnt = lax.fori_loop(0, NK // 16, count_body, zeros16)
        padded = jnp.bitwise_and(cnt + (BM - 1), jnp.int32(-BM))
        ends = plsc.cumsum(padded)
        offs = ends - padded
        run_v[...] = offs

        for bv in range(NBP // 16):
            bexp_v[pl.ds(bv * 16, 16)] = zeros16

        def z_body(i, _):
            src_v[pl.ds(i * 16, 16)] = zeros16
            return 0

        lax.fori_loop(0, P // 16, z_body, 0)

        def rank_body(i, _):
            v = e_v[pl.ds(i * 16, 16)]
            base = plsc.load_gather(run_v, [v])
            rank = jnp.zeros((16,), jnp.int32)
            run = run_v[...]
            for e in range(E):
                msk = v == e
                cs = plsc.cumsum(jnp.where(msk, 1, 0))
                rank = jnp.where(msk, cs - 1, rank)
                run = run + jnp.where(lanes == e, jnp.sum(jnp.where(msk, 1, 0)), 0)
            run_v[...] = run
            dest = base + rank
            dest_v[pl.ds(i * 16, 16)] = dest
            tok = jnp.right_shift(lanes + i * 16, 1)
            plsc.store_scatter(src_v, [dest], tok)
            plsc.store_scatter(bexp_v, [jnp.right_shift(dest, BM.bit_length() - 1)], v)
            return 0

        lax.fori_loop(0, NK // 16, rank_body, 0)
        pltpu.sync_copy(dest_v, dest_hbm)
        pltpu.sync_copy(src_v, src_hbm)
        pltpu.sync_copy(bexp_v, bexp_hbm)


def _dispatch(e_flat):
    mesh = plsc.VectorSubcoreMesh(core_axis_name="c", subcore_axis_name="s")
    f = functools.partial(
        pl.kernel, mesh=mesh,
        compiler_params=pltpu.CompilerParams(needs_layout_passes=False),
        out_type=(jax.ShapeDtypeStruct((NK,), jnp.int32),
                  jax.ShapeDtypeStruct((P,), jnp.int32),
                  jax.ShapeDtypeStruct((NBP,), jnp.int32)),
        scratch_types=[
            pltpu.VMEM((NK,), jnp.int32),
            pltpu.VMEM((NK,), jnp.int32),
            pltpu.VMEM((P,), jnp.int32),
            pltpu.VMEM((NBP,), jnp.int32),
            pltpu.VMEM((16,), jnp.int32),
            pltpu.SemaphoreType.DMA,
        ],
    )(_dispatch_body)
    return f(e_flat)


GCH = 64
GNCH = (P // NW) // GCH


def _gather_body(x_hbm, src_hbm, xpad_hbm, idx_v, rows_v, sem):
    cid = lax.axis_index("c")
    sid = lax.axis_index("s")
    wid = sid * NC + cid
    base = wid * (P // NW)
    for ch in range(GNCH):
        pltpu.sync_copy(src_hbm.at[pl.ds(base + GCH * ch, GCH)], idx_v)
        pltpu.async_copy(x_hbm.at[idx_v], rows_v, sem).wait()
        pltpu.sync_copy(rows_v, xpad_hbm.at[pl.ds(base + GCH * ch, GCH)])


def _gather(xf, src):
    mesh = plsc.VectorSubcoreMesh(core_axis_name="c", subcore_axis_name="s")
    f = functools.partial(
        pl.kernel, mesh=mesh,
        compiler_params=pltpu.CompilerParams(needs_layout_passes=False),
        out_type=jax.ShapeDtypeStruct((P, D), jnp.float32),
        scratch_types=[
            pltpu.VMEM((GCH,), jnp.int32),
            pltpu.VMEM((GCH, D), jnp.float32),
            pltpu.SemaphoreType.DMA,
        ],
    )(_gather_body)
    return f(xf, src)


HT2 = 256
NH2 = H // HT2


def _dmlp_body(bexp_ref, xp_ref, w1_ref, w2_ref, out_ref):
    h = pl.program_id(0)
    b = pl.program_id(1)
    xb = xp_ref[pl.ds(b * BM, BM), :]
    hb = _gelu_exact(
        jax.lax.dot_general(xb, w1_ref[0], (((1,), (0,)), ((), ())),
                            preferred_element_type=jnp.float32))
    contrib = jax.lax.dot_general(hb, w2_ref[0], (((1,), (0,)), ((), ())),
                                  preferred_element_type=jnp.float32)

    @pl.when(h == 0)
    def _():
        out_ref[pl.ds(b * BM, BM), :] = contrib

    @pl.when(h != 0)
    def _():
        out_ref[pl.ds(b * BM, BM), :] = out_ref[pl.ds(b * BM, BM), :] + contrib


def _dmlp(bexp, xpad, W1, W2, interpret=False):
    grid_spec = pltpu.PrefetchScalarGridSpec(
        num_scalar_prefetch=1,
        grid=(NH2, NB),
        in_specs=[
            pl.BlockSpec((P, D), lambda h, b, be: (0, 0)),
            pl.BlockSpec((1, D, HT2), lambda h, b, be: (be[b], 0, h)),
            pl.BlockSpec((1, HT2, D), lambda h, b, be: (be[b], h, 0)),
        ],
        out_specs=pl.BlockSpec((P, D), lambda h, b, be: (0, 0)),
    )
    return pl.pallas_call(
        _dmlp_body, grid_spec=grid_spec,
        out_shape=jax.ShapeDtypeStruct((P, D), jnp.float32),
        interpret=interpret,
    )(bexp, xpad, W1, W2)


CTOK = 32
CNCH = (N // NW) // CTOK


def _combine_body(ypad_hbm, dest_hbm, g_hbm, y_hbm,
                  didx_v, g_v, rows_v, y_v, sem):
    cid = lax.axis_index("c")
    sid = lax.axis_index("s")
    wid = sid * NC + cid
    tpt = N // NW
    for ch in range(CNCH):
        tb = wid * tpt + CTOK * ch
        pltpu.sync_copy(dest_hbm.at[pl.ds(2 * tb, 2 * CTOK)], didx_v)
        pltpu.sync_copy(g_hbm.at[pl.ds(2 * tb, 2 * CTOK)], g_v)
        pltpu.async_copy(ypad_hbm.at[didx_v], rows_v, sem).wait()

        def tbody(t, _):
            ga = plsc.load_gather(g_v, [jnp.full((16,), 2 * t, jnp.int32)])
            gb = plsc.load_gather(g_v, [jnp.full((16,), 2 * t + 1, jnp.int32)])

            def vbody(vi, __):
                sl = pl.ds(vi * 16, 16)
                y_v[t, sl] = ga * rows_v[2 * t, sl] + gb * rows_v[2 * t + 1, sl]
                return 0

            return lax.fori_loop(0, D // 16, vbody, 0)

        lax.fori_loop(0, CTOK, tbody, 0)
        pltpu.sync_copy(y_v, y_hbm.at[pl.ds(tb, CTOK)])


def _combine(ypad, dest, g_flat):
    mesh = plsc.VectorSubcoreMesh(core_axis_name="c", subcore_axis_name="s")
    f = functools.partial(
        pl.kernel, mesh=mesh,
        compiler_params=pltpu.CompilerParams(needs_layout_passes=False),
        out_type=jax.ShapeDtypeStruct((N, D), jnp.float32),
        scratch_types=[
            pltpu.VMEM((2 * CTOK,), jnp.int32),
            pltpu.VMEM((2 * CTOK,), jnp.float32),
            pltpu.VMEM((2 * CTOK, D), jnp.float32),
            pltpu.VMEM((CTOK, D), jnp.float32),
            pltpu.SemaphoreType.DMA,
        ],
    )(_combine_body)
    return f(ypad, dest, g_flat)


def kernel(x, Wg, W1, W2):
    Bs, Ss, Dm = x.shape
    xf = x.reshape(-1, Dm)
    g2, e2 = _router(xf, Wg)
    e_flat = e2.reshape(-1)
    g_flat = g2.reshape(-1)
    dest, src, bexp = _dispatch(e_flat)
    xpad = _gather(xf, src)
    ypad = _dmlp(bexp, xpad, W1, W2)
    y = _combine(ypad, dest, g_flat)
    return y.reshape(Bs, Ss, Dm)


# speedup vs baseline: 1.5403x; 1.5403x over previous
import functools

import jax
import jax.numpy as jnp
from jax import lax
from jax.experimental import pallas as pl
from jax.experimental.pallas import tpu as pltpu
from jax.experimental.pallas import tpu_sc as plsc

E = 8
K = 2
D = 1024
H = 4096
N = 2048
NK = N * K
BM = 256
P = NK + E * BM
NB = P // BM
NBP = 32
HT1 = 512
NH1 = H // HT1

NC = 2
NS = 16
NW = NC * NS


def _gelu_exact(v):
    return v * 0.5 * (1.0 + jax.lax.erf(v * (2.0 ** -0.5)))


def _router_body(x_ref, wg_ref, g_ref, e_ref):
    xv = x_ref[...]
    wg = wg_ref[0:E, :]
    logits = jax.lax.dot_general(xv, wg, (((1,), (1,)), ((), ())),
                                 preferred_element_type=jnp.float32)
    m = jnp.max(logits, axis=1, keepdims=True)
    p = jnp.exp(logits - m)
    p = p / jnp.sum(p, axis=1, keepdims=True)
    cols = jax.lax.broadcasted_iota(jnp.int32, p.shape, 1)
    m1 = jnp.max(p, axis=1, keepdims=True)
    i1 = jnp.min(jnp.where(p == m1, cols, E), axis=1, keepdims=True)
    mask1 = cols == i1
    p2 = jnp.where(mask1, -1.0, p)
    m2 = jnp.max(p2, axis=1, keepdims=True)
    i2 = jnp.min(jnp.where(p2 == m2, cols, E), axis=1, keepdims=True)
    g_ref[...] = jnp.concatenate([m1, m2], axis=1)
    e_ref[...] = jnp.concatenate([i1, i2], axis=1)


def _router(xf, Wg, interpret=False):
    return pl.pallas_call(
        _router_body,
        out_shape=(jax.ShapeDtypeStruct((N, K), jnp.float32),
                   jax.ShapeDtypeStruct((N, K), jnp.int32)),
        interpret=interpret,
    )(xf, Wg)


def _dispatch_body(e_hbm, dest_hbm, src_hbm, bexp_hbm,
                   e_v, dest_v, src_v, bexp_v, run_v, sem):
    cid = lax.axis_index("c")
    sid = lax.axis_index("s")

    @pl.when(jnp.logical_and(cid == 0, sid == 0))
    def _():
        pltpu.sync_copy(e_hbm, e_v)
        lanes = jax.lax.broadcasted_iota(jnp.int32, (16,), 0)
        zeros16 = jnp.zeros((16,), jnp.int32)

        def count_body(i, cnt):
            v = e_v[pl.ds(i * 16, 16)]
            for e in range(E):
                ce = jnp.sum(jnp.where(v == e, 1, 0))
                cnt = cnt + jnp.where(lanes == e, ce, 0)
            return cnt

        cnt = lax.fori_loop(0, NK // 16, count_body, zeros16)
        padded = jnp.bitwise_and(cnt + (BM - 1), jnp.int32(-BM))
        ends = plsc.cumsum(padded)
        offs = ends - padded
        run_v[...] = offs

        for bv in range(NBP // 16):
            bexp_v[pl.ds(bv * 16, 16)] = zeros16

        def z_body(i, _):
            src_v[pl.ds(i * 16, 16)] = zeros16
            return 0

        lax.fori_loop(0, P // 16, z_body, 0)

        def rank_body(i, _):
            v = e_v[pl.ds(i * 16, 16)]
            base = plsc.load_gather(run_v, [v])
            rank = jnp.zeros((16,), jnp.int32)
            run = run_v[...]
            for e in range(E):
                msk = v == e
                cs = plsc.cumsum(jnp.where(msk, 1, 0))
                rank = jnp.where(msk, cs - 1, rank)
                run = run + jnp.where(lanes == e, jnp.sum(jnp.where(msk, 1, 0)), 0)
            run_v[...] = run
            dest = base + rank
            dest_v[pl.ds(i * 16, 16)] = dest
            tok = jnp.right_shift(lanes + i * 16, 1)
            plsc.store_scatter(src_v, [dest], tok)
            plsc.store_scatter(bexp_v, [jnp.right_shift(dest, BM.bit_length() - 1)], v)
            return 0

        lax.fori_loop(0, NK // 16, rank_body, 0)
        pltpu.sync_copy(dest_v, dest_hbm)
        pltpu.sync_copy(src_v, src_hbm)
        pltpu.sync_copy(bexp_v, bexp_hbm)


def _dispatch(e_flat):
    mesh = plsc.VectorSubcoreMesh(core_axis_name="c", subcore_axis_name="s")
    f = functools.partial(
        pl.kernel, mesh=mesh,
        compiler_params=pltpu.CompilerParams(needs_layout_passes=False),
        out_type=(jax.ShapeDtypeStruct((NK,), jnp.int32),
                  jax.ShapeDtypeStruct((P,), jnp.int32),
                  jax.ShapeDtypeStruct((NBP,), jnp.int32)),
        scratch_types=[
            pltpu.VMEM((NK,), jnp.int32),
            pltpu.VMEM((NK,), jnp.int32),
            pltpu.VMEM((P,), jnp.int32),
            pltpu.VMEM((NBP,), jnp.int32),
            pltpu.VMEM((16,), jnp.int32),
            pltpu.SemaphoreType.DMA,
        ],
    )(_dispatch_body)
    return f(e_flat)


GCH = 48
GNCH = (P // NW) // GCH


def _gather_body(x_hbm, src_hbm, xpad_hbm, idx0, idx1, rows0, rows1, sem):
    cid = lax.axis_index("c")
    sid = lax.axis_index("s")
    wid = sid * NC + cid
    base = wid * (P // NW)
    idx = (idx0, idx1)
    rows = (rows0, rows1)
    copies = [None, None]
    for ch in range(2):
        pltpu.sync_copy(src_hbm.at[pl.ds(base + GCH * ch, GCH)], idx[ch])
        copies[ch] = pltpu.async_copy(x_hbm.at[idx[ch]], rows[ch], sem)
    for ch in range(GNCH):
        s = ch % 2
        copies[s].wait()
        pltpu.sync_copy(rows[s], xpad_hbm.at[pl.ds(base + GCH * ch, GCH)])
        if ch + 2 < GNCH:
            pltpu.sync_copy(src_hbm.at[pl.ds(base + GCH * (ch + 2), GCH)],
                            idx[s])
            copies[s] = pltpu.async_copy(x_hbm.at[idx[s]], rows[s], sem)


def _gather(xf, src):
    mesh = plsc.VectorSubcoreMesh(core_axis_name="c", subcore_axis_name="s")
    f = functools.partial(
        pl.kernel, mesh=mesh,
        compiler_params=pltpu.CompilerParams(needs_layout_passes=False),
        out_type=jax.ShapeDtypeStruct((P, D), jnp.float32),
        scratch_types=[
            pltpu.VMEM((GCH,), jnp.int32),
            pltpu.VMEM((GCH,), jnp.int32),
            pltpu.VMEM((GCH, D), jnp.float32),
            pltpu.VMEM((GCH, D), jnp.float32),
            pltpu.SemaphoreType.DMA,
        ],
    )(_gather_body)
    return f(xf, src)


def _d1_body(bexp_ref, xp_ref, w1_ref, hid_ref):
    hid_ref[...] = _gelu_exact(
        jax.lax.dot_general(xp_ref[...], w1_ref[0], (((1,), (0,)), ((), ())),
                            preferred_element_type=jnp.float32))


def _d1(bexp, xpad, W1, interpret=False):
    grid_spec = pltpu.PrefetchScalarGridSpec(
        num_scalar_prefetch=1,
        grid=(NB,),
        in_specs=[
            pl.BlockSpec((BM, D), lambda b, be: (b, 0)),
            pl.BlockSpec((1, D, H), lambda b, be: (be[b], 0, 0)),
        ],
        out_specs=pl.BlockSpec((BM, H), lambda b, be: (b, 0)),
    )
    return pl.pallas_call(
        _d1_body, grid_spec=grid_spec,
        out_shape=jax.ShapeDtypeStruct((P, H), jnp.float32),
        interpret=interpret,
    )(bexp, xpad, W1)


def _d2_body(bexp_ref, hid_ref, w2_ref, yp_ref):
    yp_ref[...] = jax.lax.dot_general(
        hid_ref[...], w2_ref[0], (((1,), (0,)), ((), ())),
        preferred_element_type=jnp.float32)


def _d2(bexp, hid, W2, interpret=False):
    grid_spec = pltpu.PrefetchScalarGridSpec(
        num_scalar_prefetch=1,
        grid=(NB,),
        in_specs=[
            pl.BlockSpec((BM, H), lambda b, be: (b, 0)),
            pl.BlockSpec((1, H, D), lambda b, be: (be[b], 0, 0)),
        ],
        out_specs=pl.BlockSpec((BM, D), lambda b, be: (b, 0)),
    )
    return pl.pallas_call(
        _d2_body, grid_spec=grid_spec,
        out_shape=jax.ShapeDtypeStruct((P, D), jnp.float32),
        interpret=interpret,
    )(bexp, hid, W2)


CTOK = 32
CNCH = (N // NW) // CTOK


def _combine_body(ypad_hbm, dest_hbm, g_hbm, y_hbm,
                  didx_v, g_v, rows_v, y_v, sem):
    cid = lax.axis_index("c")
    sid = lax.axis_index("s")
    wid = sid * NC + cid
    tpt = N // NW
    for ch in range(CNCH):
        tb = wid * tpt + CTOK * ch
        pltpu.sync_copy(dest_hbm.at[pl.ds(2 * tb, 2 * CTOK)], didx_v)
        pltpu.sync_copy(g_hbm.at[pl.ds(2 * tb, 2 * CTOK)], g_v)
        pltpu.async_copy(ypad_hbm.at[didx_v], rows_v, sem).wait()

        def tbody(t, _):
            ga = plsc.load_gather(g_v, [jnp.full((16,), 2 * t, jnp.int32)])
            gb = plsc.load_gather(g_v, [jnp.full((16,), 2 * t + 1, jnp.int32)])

            def vbody(vi, __):
                sl = pl.ds(vi * 16, 16)
                y_v[t, sl] = ga * rows_v[2 * t, sl] + gb * rows_v[2 * t + 1, sl]
                return 0

            return lax.fori_loop(0, D // 16, vbody, 0)

        lax.fori_loop(0, CTOK, tbody, 0)
        pltpu.sync_copy(y_v, y_hbm.at[pl.ds(tb, CTOK)])


def _combine(ypad, dest, g_flat):
    mesh = plsc.VectorSubcoreMesh(core_axis_name="c", subcore_axis_name="s")
    f = functools.partial(
        pl.kernel, mesh=mesh,
        compiler_params=pltpu.CompilerParams(needs_layout_passes=False),
        out_type=jax.ShapeDtypeStruct((N, D), jnp.float32),
        scratch_types=[
            pltpu.VMEM((2 * CTOK,), jnp.int32),
            pltpu.VMEM((2 * CTOK,), jnp.float32),
            pltpu.VMEM((2 * CTOK, D), jnp.float32),
            pltpu.VMEM((CTOK, D), jnp.float32),
            pltpu.SemaphoreType.DMA,
        ],
    )(_combine_body)
    return f(ypad, dest, g_flat)


def kernel(x, Wg, W1, W2):
    Bs, Ss, Dm = x.shape
    xf = x.reshape(-1, Dm)
    g2, e2 = _router(xf, Wg)
    e_flat = e2.reshape(-1)
    g_flat = g2.reshape(-1)
    dest, src, bexp = _dispatch(e_flat)
    xpad = _gather(xf, src)
    hid = _d1(bexp, xpad, W1)
    ypad = _d2(bexp, hid, W2)
    y = _combine(ypad, dest, g_flat)
    return y.reshape(Bs, Ss, Dm)
